# baseline (speedup 1.0000x reference)
import jax
import jax.numpy as jnp
from jax.experimental import pallas as pl
from jax.sharding import Mesh, PartitionSpec as P

try:
    from jax import shard_map as _shard_map

    def _smap(f, mesh, in_specs, out_specs):
        return _shard_map(f, mesh=mesh, in_specs=in_specs, out_specs=out_specs,
                          check_vma=False)
except ImportError:
    from jax.experimental.shard_map import shard_map as _shard_map_old

    def _smap(f, mesh, in_specs, out_specs):
        return _shard_map_old(f, mesh=mesh, in_specs=in_specs,
                              out_specs=out_specs, check_rep=False)

_TOKENS = 2048
_D_IN = 768
_BASE = 128
_BT = 256
_NDEV = 2
_LTOK = _TOKENS // _NDEV


def _pkm_kernel(x_ref, w_ref, b_ref, o_ref):
    xs = jax.lax.dot_general(
        x_ref[:], w_ref[:],
        (((1,), (1,)), ((), ())),
        preferred_element_type=jnp.float32,
    ) + b_ref[:]
    x1 = xs[:, :_BASE]
    x2 = xs[:, _BASE:]
    for i in range(_BASE):
        o_ref[:, i * _BASE:(i + 1) * _BASE] = x1[:, i:i + 1] + x2


def _expand_local(x, W, b2):
    return pl.pallas_call(
        _pkm_kernel,
        grid=(_LTOK // _BT,),
        in_specs=[
            pl.BlockSpec((_BT, _D_IN), lambda t: (t, 0)),
            pl.BlockSpec((2 * _BASE, _D_IN), lambda t: (0, 0)),
            pl.BlockSpec((1, 2 * _BASE), lambda t: (0, 0)),
        ],
        out_specs=pl.BlockSpec((_BT, _BASE * _BASE), lambda t: (t, 0)),
        out_shape=jax.ShapeDtypeStruct((_LTOK, _BASE * _BASE), jnp.float32),
    )(x, W, b2)


def kernel(x, W, b):
    b2 = b.reshape(1, 2 * _BASE)
    devs = jax.devices()
    if len(devs) >= _NDEV:
        mesh = Mesh(devs[:_NDEV], ("d",))
        f = _smap(
            _expand_local, mesh,
            (P("d", None), P(None, None), P(None, None)),
            P("d", None),
        )
        return f(x, W, b2)
    return _expand_local(x, W, b2)


# speedup vs baseline: 9.2989x; 9.2989x over previous
import jax
import jax.numpy as jnp
from jax.experimental import pallas as pl
import jax.experimental.pallas.tpu as pltpu

_TOKENS = 2048
_D_IN = 768
_BASE = 128
_BT = 256


def _pkm_kernel(x_ref, w_ref, b_ref, o_ref):
    xs = jax.lax.dot_general(
        x_ref[:], w_ref[:],
        (((1,), (1,)), ((), ())),
        preferred_element_type=jnp.float32,
    ) + b_ref[:]
    x1 = xs[:, :_BASE]
    x2 = xs[:, _BASE:]
    for i in range(_BASE):
        o_ref[:, i * _BASE:(i + 1) * _BASE] = x1[:, i:i + 1] + x2


def kernel(x, W, b):
    b2 = b.reshape(1, 2 * _BASE)
    return pl.pallas_call(
        _pkm_kernel,
        grid=(_TOKENS // _BT,),
        in_specs=[
            pl.BlockSpec((_BT, _D_IN), lambda t: (t, 0)),
            pl.BlockSpec((2 * _BASE, _D_IN), lambda t: (0, 0)),
            pl.BlockSpec((1, 2 * _BASE), lambda t: (0, 0)),
        ],
        out_specs=pl.BlockSpec((_BT, _BASE * _BASE), lambda t: (t, 0)),
        out_shape=jax.ShapeDtypeStruct((_TOKENS, _BASE * _BASE), jnp.float32),
        compiler_params=pltpu.CompilerParams(
            dimension_semantics=("parallel",),
        ),
    )(x, W, b2)
